# baseline (speedup 1.0000x reference)
import functools

import jax
import jax.numpy as jnp
from jax import lax
from jax.experimental import pallas as pl
from jax.experimental.pallas import tpu as pltpu
from jax.experimental.pallas import tpu_sc as plsc

B, N, D, R = 128, 1024, 384, 512
TOTAL = B * N
H = D // 2

_info = plsc.get_sparse_core_info()
NC, NS, L = _info.num_cores, _info.num_subcores, _info.num_lanes
NW = NC * NS
W = TOTAL // NW
C = 32
NCHUNK = W // C
NBUF = 2


def _sc_posadd(x2, coords2, t1):
    mesh = plsc.VectorSubcoreMesh(core_axis_name="c", subcore_axis_name="s")

    @functools.partial(
        pl.kernel,
        mesh=mesh,
        out_type=jax.ShapeDtypeStruct((TOTAL, D), jnp.float32),
        scratch_types=(
            [pltpu.VMEM((R * H,), jnp.float32)]
            + [pltpu.VMEM((C, D), jnp.float32)] * NBUF
            + [pltpu.VMEM((C,), jnp.int32)] * NBUF
            + [pltpu.VMEM((C,), jnp.int32)] * NBUF
            + [pltpu.SemaphoreType.DMA] * (4 * NBUF)
        ),
    )
    def k(x_hbm, yi_hbm, xi_hbm, t1_hbm, out_hbm, t1_v, *bufs):
        xbufs = bufs[0:NBUF]
        ybufs = bufs[NBUF:2 * NBUF]
        xibufs = bufs[2 * NBUF:3 * NBUF]
        in_s = bufs[3 * NBUF:4 * NBUF]
        yi_s = bufs[4 * NBUF:5 * NBUF]
        xi_s = bufs[5 * NBUF:6 * NBUF]
        out_s = bufs[6 * NBUF:7 * NBUF]
        wid = lax.axis_index("s") * NC + lax.axis_index("c")
        base = wid * W
        pltpu.sync_copy(t1_hbm, t1_v)

        def start_in(c, b):
            return pltpu.async_copy(x_hbm.at[pl.ds(base + c * C, C)], xbufs[b], in_s[b])

        def start_ci(c, b):
            pltpu.async_copy(yi_hbm.at[pl.ds(base + c * C, C)], ybufs[b], yi_s[b])
            pltpu.async_copy(xi_hbm.at[pl.ds(base + c * C, C)], xibufs[b], xi_s[b])

        def wait_in(c, b):
            pltpu.make_async_copy(
                x_hbm.at[pl.ds(base + c * C, C)], xbufs[b], in_s[b]).wait()

        def wait_ci(c, b):
            pltpu.make_async_copy(
                yi_hbm.at[pl.ds(base + c * C, C)], ybufs[b], yi_s[b]).wait()
            pltpu.make_async_copy(
                xi_hbm.at[pl.ds(base + c * C, C)], xibufs[b], xi_s[b]).wait()

        def start_out(c, b):
            return pltpu.async_copy(
                xbufs[b], out_hbm.at[pl.ds(base + c * C, C)], out_s[b])

        def wait_out(c, b):
            pltpu.make_async_copy(
                xbufs[b], out_hbm.at[pl.ds(base + c * C, C)], out_s[b]).wait()

        def add_chunk(b):
            xb, yb, xib = xbufs[b], ybufs[b], xibufs[b]

            def add_blk(kk, carry):
                yv = yb[pl.ds(kk * L, L)]
                xv = xib[pl.ds(kk * L, L)]
                for lane in range(L):
                    r = kk * L + lane
                    yrow = yv[lane]
                    xrow = xv[lane]
                    for j in range(H // L):
                        s = pl.ds(j * L, L)
                        plsc.addupdate(xb.at[r, s], t1_v[pl.ds(yrow * H + j * L, L)])
                    for j in range(H // L):
                        s = pl.ds(j * L, L)
                        s2 = pl.ds(H + j * L, L)
                        plsc.addupdate(xb.at[r, s2], t1_v[pl.ds(xrow * H + j * L, L)])
                return carry

            lax.fori_loop(0, C // L, add_blk, 0)

        for c0 in range(NBUF):
            start_in(c0, c0)
            start_ci(c0, c0)

        def pair_body(p, carry):
            for b in range(NBUF):
                c = NBUF * p + b
                wait_in(c, b)
                wait_ci(c, b)
                add_chunk(b)
                start_out(c, b).wait()
                start_in(c + NBUF, b)
                start_ci(c + NBUF, b)
            return carry

        lax.fori_loop(0, (NCHUNK - NBUF) // NBUF, pair_body, 0)

        for c in range(NCHUNK - NBUF, NCHUNK):
            b = c % NBUF
            wait_in(c, b)
            wait_ci(c, b)
            add_chunk(b)
            start_out(c, b).wait()

    return k(x2, coords2[0], coords2[1], t1)


def kernel(x, offgrid_coords, pos_table):
    x2 = x.reshape(TOTAL, D)
    yi = offgrid_coords[..., 1].reshape(TOTAL)
    xi = offgrid_coords[..., 0].reshape(TOTAL)
    coords2 = jnp.stack([yi, xi], axis=0)
    t1 = pos_table[0, :, H:].reshape(-1)
    out = _sc_posadd(x2, coords2, t1)
    return out.reshape(B, N, D)


# speedup vs baseline: 1.5193x; 1.5193x over previous
import functools

import jax
import jax.numpy as jnp
from jax import lax
from jax.experimental import pallas as pl
from jax.experimental.pallas import tpu as pltpu
from jax.experimental.pallas import tpu_sc as plsc

B, N, D, R = 128, 1024, 384, 512
TOTAL = B * N
H = D // 2
HP = H // 2
HPAD = 128

_info = plsc.get_sparse_core_info()
NC, NS, L = _info.num_cores, _info.num_subcores, _info.num_lanes
NW = NC * NS
W = TOTAL // NW
C = 32
NCHUNK = W // C
NBUF = 4
K = NBUF - 1
_HIMASK = jnp.int32(-65536)


def _sc_posadd(x2, yi, xi, t1p):
    mesh = plsc.VectorSubcoreMesh(core_axis_name="c", subcore_axis_name="s")

    @functools.partial(
        pl.kernel,
        mesh=mesh,
        out_type=jax.ShapeDtypeStruct((TOTAL, D), jnp.float32),
        scratch_types=(
            [pltpu.VMEM((W,), jnp.int32)] * 2
            + [pltpu.VMEM((C, D), jnp.float32)] * NBUF
            + [pltpu.VMEM((C, HPAD), jnp.int32)] * NBUF
            + [pltpu.VMEM((C, HPAD), jnp.int32)] * NBUF
            + [pltpu.SemaphoreType.DMA] * (4 * NBUF)
        ),
    )
    def k(x_hbm, yi_hbm, xi_hbm, t1_hbm, out_hbm, yi_v, xi_v, *bufs):
        xbufs = bufs[0:NBUF]
        rby = bufs[NBUF:2 * NBUF]
        rbx = bufs[2 * NBUF:3 * NBUF]
        in_s = bufs[3 * NBUF:4 * NBUF]
        gy_s = bufs[4 * NBUF:5 * NBUF]
        gx_s = bufs[5 * NBUF:6 * NBUF]
        out_s = bufs[6 * NBUF:7 * NBUF]
        wid = lax.axis_index("s") * NC + lax.axis_index("c")
        base = wid * W
        pltpu.sync_copy(yi_hbm.at[pl.ds(base, W)], yi_v)
        pltpu.sync_copy(xi_hbm.at[pl.ds(base, W)], xi_v)

        def start_in(c, b):
            return pltpu.async_copy(x_hbm.at[pl.ds(base + c * C, C)], xbufs[b], in_s[b])

        def start_ga(c, b):
            pltpu.async_copy(t1_hbm.at[yi_v.at[pl.ds(c * C, C)]], rby[b], gy_s[b])
            pltpu.async_copy(t1_hbm.at[xi_v.at[pl.ds(c * C, C)]], rbx[b], gx_s[b])

        def wait_in(c, b):
            pltpu.make_async_copy(
                x_hbm.at[pl.ds(base + c * C, C)], xbufs[b], in_s[b]).wait()

        def wait_ga(c, b):
            pltpu.make_async_copy(
                t1_hbm.at[yi_v.at[pl.ds(c * C, C)]], rby[b], gy_s[b]).wait()
            pltpu.make_async_copy(
                t1_hbm.at[xi_v.at[pl.ds(c * C, C)]], rbx[b], gx_s[b]).wait()

        def start_out(c, b):
            return pltpu.async_copy(
                xbufs[b], out_hbm.at[pl.ds(base + c * C, C)], out_s[b])

        def wait_out(c, b):
            pltpu.make_async_copy(
                xbufs[b], out_hbm.at[pl.ds(base + c * C, C)], out_s[b]).wait()

        def add_chunk(b):
            xb, ry, rx = xbufs[b], rby[b], rbx[b]

            def add_row(r, carry):
                for half, rbuf in ((0, ry), (1, rx)):
                    for j in range(HP // L):
                        rv = rbuf[r, pl.ds(j * L, L)]
                        lo = lax.bitcast_convert_type(rv << 16, jnp.float32)
                        hi = lax.bitcast_convert_type(rv & (-65536), jnp.float32)
                        off = half * H + j * 2 * L
                        plsc.addupdate(xb.at[r, pl.ds(off, L)], lo)
                        plsc.addupdate(xb.at[r, pl.ds(off + L, L)], hi)
                return carry

            lax.fori_loop(0, C, add_row, 0)

        for c0 in range(K):
            start_in(c0, c0)
            start_ga(c0, c0)

        wait_in(0, 0)
        wait_ga(0, 0)
        add_chunk(0)
        start_out(0, 0)
        start_in(K, K % NBUF)
        start_ga(K, K % NBUF)

        def quad_body(q, carry):
            for j in range(NBUF):
                c = NBUF * q + 1 + j
                b = (1 + j) % NBUF
                wait_in(c, b)
                wait_ga(c, b)
                add_chunk(b)
                start_out(c, b)
                bp = (b + K) % NBUF
                wait_out(c - 1, bp)
                start_in(c + K, bp)
                start_ga(c + K, bp)
            return carry

        lax.fori_loop(0, (NCHUNK - NBUF) // NBUF, quad_body, 0)

        for c in range(NCHUNK - K, NCHUNK):
            b = c % NBUF
            wait_in(c, b)
            wait_ga(c, b)
            add_chunk(b)
            start_out(c, b)
        for c in range(NCHUNK - NBUF, NCHUNK):
            wait_out(c, c % NBUF)

    return k(x2, yi, xi, t1p)


def _pack_t1(pos_table):
    t1 = pos_table[0, :, H:]
    t1r = t1.reshape(R, HP // L, 2, L)
    lo = t1r[:, :, 0, :].astype(jnp.bfloat16)
    hi = t1r[:, :, 1, :].astype(jnp.bfloat16)
    lo_u = lax.bitcast_convert_type(lo, jnp.uint16).astype(jnp.uint32)
    hi_u = lax.bitcast_convert_type(hi, jnp.uint16).astype(jnp.uint32)
    word = lo_u | (hi_u << 16)
    t1p = lax.bitcast_convert_type(word, jnp.int32).reshape(R, HP)
    return jnp.pad(t1p, ((0, 0), (0, HPAD - HP)))


def kernel(x, offgrid_coords, pos_table):
    x2 = x.reshape(TOTAL, D)
    yi = offgrid_coords[..., 1].reshape(TOTAL)
    xi = offgrid_coords[..., 0].reshape(TOTAL)
    t1p = _pack_t1(pos_table)
    out = _sc_posadd(x2, yi, xi, t1p)
    return out.reshape(B, N, D)


# speedup vs baseline: 1.9944x; 1.3127x over previous
import functools

import jax
import jax.numpy as jnp
from jax import lax
from jax.experimental import pallas as pl
from jax.experimental.pallas import tpu as pltpu
from jax.experimental.pallas import tpu_sc as plsc

B, N, D, R = 128, 1024, 384, 512
TOTAL = B * N
V = R * R

_info = plsc.get_sparse_core_info()
NC, NS, L = _info.num_cores, _info.num_subcores, _info.num_lanes
NW = NC * NS
W = TOTAL // NW
C = 32
NCHUNK = W // C
NBUF = 4
K = NBUF - 1


def _sc_gather_add(x2, xi, yi, table2):
    mesh = plsc.VectorSubcoreMesh(core_axis_name="c", subcore_axis_name="s")

    @functools.partial(
        pl.kernel,
        mesh=mesh,
        out_type=jax.ShapeDtypeStruct((TOTAL, D), jnp.float32),
        scratch_types=(
            [pltpu.VMEM((W,), jnp.int32)] * 3
            + [pltpu.VMEM((C, D), jnp.float32)] * NBUF
            + [pltpu.VMEM((C, D), jnp.float32)] * NBUF
            + [pltpu.SemaphoreType.DMA] * (3 * NBUF)
        ),
    )
    def k(x_hbm, xi_hbm, yi_hbm, tab_hbm, out_hbm, xi_v, yi_v, idx_v, *bufs):
        xbufs = bufs[0:NBUF]
        rbufs = bufs[NBUF:2 * NBUF]
        in_s = bufs[2 * NBUF:3 * NBUF]
        ga_s = bufs[3 * NBUF:4 * NBUF]
        out_s = bufs[4 * NBUF:5 * NBUF]
        wid = lax.axis_index("s") * NC + lax.axis_index("c")
        base = wid * W
        pltpu.sync_copy(xi_hbm.at[pl.ds(base, W)], xi_v)
        pltpu.sync_copy(yi_hbm.at[pl.ds(base, W)], yi_v)

        def compute_idx(t, carry):
            s = pl.ds(t * L, L)
            idx_v[s] = yi_v[s] * R + xi_v[s]
            return carry

        lax.fori_loop(0, W // L, compute_idx, 0)

        def start_in(c, b):
            return pltpu.async_copy(x_hbm.at[pl.ds(base + c * C, C)], xbufs[b], in_s[b])

        def start_ga(c, b):
            return pltpu.async_copy(
                tab_hbm.at[idx_v.at[pl.ds(c * C, C)]], rbufs[b], ga_s[b])

        def wait_in(c, b):
            pltpu.make_async_copy(
                x_hbm.at[pl.ds(base + c * C, C)], xbufs[b], in_s[b]).wait()

        def wait_ga(c, b):
            pltpu.make_async_copy(
                tab_hbm.at[idx_v.at[pl.ds(c * C, C)]], rbufs[b], ga_s[b]).wait()

        def start_out(c, b):
            return pltpu.async_copy(
                xbufs[b], out_hbm.at[pl.ds(base + c * C, C)], out_s[b])

        def wait_out(c, b):
            pltpu.make_async_copy(
                xbufs[b], out_hbm.at[pl.ds(base + c * C, C)], out_s[b]).wait()

        def add_chunk(b):
            xb, rb = xbufs[b], rbufs[b]

            def add_row(r, carry):
                for j in range(D // L):
                    s = pl.ds(j * L, L)
                    plsc.addupdate(xb.at[r, s], rb[r, s])
                return carry

            lax.fori_loop(0, C, add_row, 0)

        for c0 in range(K):
            start_in(c0, c0)
            start_ga(c0, c0)

        wait_in(0, 0)
        wait_ga(0, 0)
        add_chunk(0)
        start_out(0, 0)
        start_in(K, K % NBUF)
        start_ga(K, K % NBUF)

        def quad_body(q, carry):
            for j in range(NBUF):
                c = NBUF * q + 1 + j
                b = (1 + j) % NBUF
                wait_in(c, b)
                wait_ga(c, b)
                add_chunk(b)
                start_out(c, b)
                bp = (b + K) % NBUF
                wait_out(c - 1, bp)
                start_in(c + K, bp)
                start_ga(c + K, bp)
            return carry

        lax.fori_loop(0, (NCHUNK - NBUF) // NBUF, quad_body, 0)

        for c in range(NCHUNK - K, NCHUNK):
            b = c % NBUF
            wait_in(c, b)
            wait_ga(c, b)
            add_chunk(b)
            start_out(c, b)
        for c in range(NCHUNK - NBUF, NCHUNK):
            wait_out(c, c % NBUF)

    return k(x2, xi, yi, table2)


def kernel(x, offgrid_coords, pos_table):
    x2 = x.reshape(TOTAL, D)
    xi = offgrid_coords[..., 0].reshape(TOTAL)
    yi = offgrid_coords[..., 1].reshape(TOTAL)
    table2 = pos_table.reshape(V, D)
    out = _sc_gather_add(x2, xi, yi, table2)
    return out.reshape(B, N, D)
